# baseline (speedup 1.0000x reference)
import functools

import jax
import jax.numpy as jnp
from jax import lax
from jax.experimental import pallas as pl
from jax.experimental.pallas import tpu as pltpu
from jax.experimental.pallas import tpu_sc as plsc

_N = 10000
_E = 160000
_D = 256

_NT = 32
_RPT = 320
_NPAD = _NT * _RPT
_EPP = _E // _NT
_BCAP = 320
_BW = 336
_LISTS = _NT * _BW
_OCAP = _EPP + 24
_CH = 48
_NCHUNK = _BW // _CH
_DUMMY = _RPT
_PACK = 16384


def _sc_partition(row, col):
    mesh = plsc.VectorSubcoreMesh(core_axis_name="c", subcore_axis_name="s")

    @functools.partial(
        pl.kernel,
        mesh=mesh,
        out_type=(
            jax.ShapeDtypeStruct((_NT * _LISTS,), jnp.int32),
            jax.ShapeDtypeStruct((_NT * 48,), jnp.int32),
            jax.ShapeDtypeStruct((_NT * _OCAP,), jnp.int32),
        ),
        scratch_types=[
            pltpu.VMEM((_EPP + 16,), jnp.int32),
            pltpu.VMEM((_EPP + 16,), jnp.int32),
            pltpu.VMEM((_LISTS,), jnp.int32),
            pltpu.VMEM((48,), jnp.int32),
            pltpu.VMEM((_OCAP,), jnp.int32),
        ],
    )
    def part_kernel(row_hbm, col_hbm, lists_hbm, counts_hbm, ovf_hbm,
                    rowv, colv, lists_v, cnt_v, ovf_v):
        c = lax.axis_index("c")
        s = lax.axis_index("s")
        p = c * 16 + s
        lane = lax.broadcasted_iota(jnp.int32, (16,), 0)
        neg1 = jnp.full((16,), -1, jnp.int32)
        zero16 = jnp.zeros((16,), jnp.int32)

        def _fill_lists(i, _):
            lists_v[pl.ds(i * 16, 16)] = neg1
            return 0

        lax.fori_loop(0, _LISTS // 16, _fill_lists, 0)

        def _fill_ovf(i, _):
            ovf_v[pl.ds(i * 16, 16)] = neg1
            return 0

        lax.fori_loop(0, _OCAP // 16, _fill_ovf, 0)
        for i in range(3):
            cnt_v[pl.ds(i * 16, 16)] = zero16

        base = p * _EPP
        pltpu.sync_copy(row_hbm.at[pl.ds(base, _EPP)], rowv.at[pl.ds(0, _EPP)])
        pltpu.sync_copy(col_hbm.at[pl.ds(base, _EPP)], colv.at[pl.ds(0, _EPP)])
        ngroups = (_EPP + 15) // 16
        tail = (ngroups - 1) * 16
        rt = rowv[pl.ds(tail, 16)]
        ct = colv[pl.ds(tail, 16)]
        nvalid = _EPP - tail
        rowv[pl.ds(tail, 16)] = jnp.where(lane < nvalid, rt, 1)
        colv[pl.ds(tail, 16)] = jnp.where(lane < nvalid, ct, 1)

        def _group(g, _):
            r16 = rowv[pl.ds(g * 16, 16)]
            c16 = colv[pl.ds(g * 16, 16)]
            b16 = (r16 * 13108) >> 22
            local16 = r16 - b16 * _RPT
            word16 = local16 * _PACK + c16
            gword16 = r16 * _PACK + c16
            bb16 = jnp.where(r16 != c16, b16, _NT)
            for l in range(16):
                bv = bb16[l]

                @pl.when(bv < _NT)
                def _append():
                    cntv = cnt_v[pl.ds(bv, 16)][0]
                    plsc.addupdate(cnt_v.at[pl.ds(bv, 16)],
                                   jnp.where(lane == 0, 1, 0))

                    @pl.when(cntv < _BCAP)
                    def _bucket():
                        wv = word16[l]
                        plsc.addupdate(
                            lists_v.at[pl.ds(bv * _BW + cntv, 16)],
                            jnp.where(lane == 0, wv + 1, 0))

                    @pl.when(cntv >= _BCAP)
                    def _overflow():
                        gv = gword16[l]
                        ov = cnt_v[pl.ds(_NT, 16)][0]
                        plsc.addupdate(cnt_v.at[pl.ds(_NT, 16)],
                                       jnp.where(lane == 0, 1, 0))
                        plsc.addupdate(ovf_v.at[pl.ds(ov, 16)],
                                       jnp.where(lane == 0, gv + 1, 0))
            return 0

        lax.fori_loop(0, ngroups, _group, 0)

        pltpu.sync_copy(lists_v, lists_hbm.at[pl.ds(p * _LISTS, _LISTS)])
        pltpu.sync_copy(cnt_v, counts_hbm.at[pl.ds(p * 48, 48)])
        pltpu.sync_copy(ovf_v, ovf_hbm.at[pl.ds(p * _OCAP, _OCAP)])

    return part_kernel(row, col)


def _sc_aggregate(x, lists, counts, ovf):
    mesh = plsc.VectorSubcoreMesh(core_axis_name="c", subcore_axis_name="s")

    @functools.partial(
        pl.kernel,
        mesh=mesh,
        out_type=jax.ShapeDtypeStruct((_NPAD, _D), jnp.float32),
        scratch_types=[
            pltpu.VMEM((_RPT + 16, _D), jnp.float32),
            pltpu.VMEM((_CH, _D), jnp.float32),
            pltpu.VMEM((_CH,), jnp.int32),
            pltpu.VMEM((_CH,), jnp.int32),
            pltpu.VMEM((_NT * 48,), jnp.int32),
            pltpu.SemaphoreType.DMA,
        ],
    )
    def agg_kernel(x_hbm, lists_hbm, counts_hbm, ovf_hbm, out_hbm,
                   acc, rows_v, cidx_v, lbuf_v, cnts_v, sem):
        c = lax.axis_index("c")
        s = lax.axis_index("s")
        w = c * 16 + s
        lane = lax.broadcasted_iota(jnp.int32, (16,), 0)
        zf16 = jnp.zeros((16,), jnp.float32)

        def _zacc(r, _):
            for j in range(_D // 16):
                acc[r, pl.ds(j * 16, 16)] = zf16
            return 0

        lax.fori_loop(0, _RPT + 16, _zacc, 0)
        pltpu.sync_copy(counts_hbm, cnts_v)

        def _producer(p, _):
            cnt = cnts_v[pl.ds(p * 48 + w, 16)][0]
            lbase = p * _LISTS + w * _BW

            def _chunk(j, _):
                @pl.when(j * _CH < cnt)
                def _do():
                    pltpu.sync_copy(
                        lists_hbm.at[pl.ds(lbase + j * _CH, _CH)], lbuf_v)
                    dgs = []
                    for g in range(_CH // 16):
                        w16 = lbuf_v[pl.ds(g * 16, 16)]
                        sent = w16 < 0
                        cidx_v[pl.ds(g * 16, 16)] = jnp.where(
                            sent, 0, w16 & (_PACK - 1))
                        dgs.append(jnp.where(sent, _DUMMY, w16 >> 14))
                    pltpu.async_copy(x_hbm.at[cidx_v], rows_v, sem).wait()
                    for e in range(_CH):
                        dv = dgs[e // 16][e % 16]
                        for jj in range(_D // 16):
                            plsc.addupdate(
                                acc.at[dv, pl.ds(jj * 16, 16)],
                                rows_v[e, pl.ds(jj * 16, 16)])
                return 0

            lax.fori_loop(0, _NCHUNK, _chunk, 0)
            return 0

        lax.fori_loop(0, _NT, _producer, 0)

        def _ovf_producer(p, _):
            ocnt = cnts_v[pl.ds(p * 48 + _NT, 16)][0]

            @pl.when(ocnt > 0)
            def _scan():
                def _ogroup(g, _):
                    @pl.when(g * 16 < ocnt)
                    def _do():
                        pltpu.sync_copy(
                            ovf_hbm.at[pl.ds(p * _OCAP + g * 16, 16)],
                            lbuf_v.at[pl.ds(0, 16)])
                        w16 = lbuf_v[pl.ds(0, 16)]
                        r16 = w16 >> 14
                        c16 = w16 & (_PACK - 1)
                        mine = jnp.where(
                            (w16 >= 0) & (r16 >= w * _RPT)
                            & (r16 < w * _RPT + _RPT), 1, 0)
                        for l in range(16):
                            @pl.when(mine[l] == 1)
                            def _one():
                                cv = c16[l]
                                dv = r16[l] - w * _RPT
                                cidx_v[pl.ds(0, 16)] = jnp.where(
                                    lane == 0, cv, 0)
                                pltpu.async_copy(
                                    x_hbm.at[cidx_v.at[pl.ds(0, 16)]],
                                    rows_v.at[pl.ds(0, 16)], sem).wait()
                                for jj in range(_D // 16):
                                    plsc.addupdate(
                                        acc.at[dv, pl.ds(jj * 16, 16)],
                                        rows_v[0, pl.ds(jj * 16, 16)])
                    return 0

                lax.fori_loop(0, (_OCAP + 15) // 16, _ogroup, 0)
            return 0

        lax.fori_loop(0, _NT, _ovf_producer, 0)

        pltpu.sync_copy(acc.at[pl.ds(0, _RPT)],
                        out_hbm.at[pl.ds(w * _RPT, _RPT)])

    return agg_kernel(x, lists, counts, ovf)


def _tc_matmul_kernel(a_ref, x_ref, w_ref, r_ref, b_ref, o_ref):
    o_ref[...] = (
        jnp.dot(a_ref[...], w_ref[...], preferred_element_type=jnp.float32)
        + jnp.dot(x_ref[...], r_ref[...], preferred_element_type=jnp.float32)
        + b_ref[...]
    )


_BLK = 400


def _tc_matmul(agg, x, weight, root, bias):
    return pl.pallas_call(
        _tc_matmul_kernel,
        grid=(_N // _BLK,),
        in_specs=[
            pl.BlockSpec((_BLK, _D), lambda i: (i, 0)),
            pl.BlockSpec((_BLK, _D), lambda i: (i, 0)),
            pl.BlockSpec((_D, _D), lambda i: (0, 0)),
            pl.BlockSpec((_D, _D), lambda i: (0, 0)),
            pl.BlockSpec((1, _D), lambda i: (0, 0)),
        ],
        out_specs=pl.BlockSpec((_BLK, _D), lambda i: (i, 0)),
        out_shape=jax.ShapeDtypeStruct((_N, _D), jnp.float32),
    )(agg, x, weight, root, bias)


@jax.jit
def kernel(x, edge_index, weight, root, bias):
    row = edge_index[0]
    col = edge_index[1]
    lists, counts, ovf = _sc_partition(row, col)
    agg = _sc_aggregate(x, lists, counts, ovf)
    return _tc_matmul(agg[:_N], x, weight, root, bias.reshape(1, _D))


# speedup vs baseline: 1.0146x; 1.0146x over previous
import functools

import jax
import jax.numpy as jnp
from jax import lax
from jax.experimental import pallas as pl
from jax.experimental.pallas import tpu as pltpu
from jax.experimental.pallas import tpu_sc as plsc

_N = 10000
_E = 160000
_D = 256

_NT = 32
_RPT = 320
_NPAD = _NT * _RPT
_EPP = _E // _NT
_BCAP = 320
_BW = 336
_LISTS = _NT * _BW
_OCAP = _EPP + 24
_CH = 48
_NCHUNK = _BW // _CH
_DUMMY = _RPT
_PACK = 16384


def _sc_partition(row, col):
    mesh = plsc.VectorSubcoreMesh(core_axis_name="c", subcore_axis_name="s")

    @functools.partial(
        pl.kernel,
        mesh=mesh,
        out_type=(
            jax.ShapeDtypeStruct((_NT * _LISTS,), jnp.int32),
            jax.ShapeDtypeStruct((_NT * 48,), jnp.int32),
            jax.ShapeDtypeStruct((_NT * _OCAP,), jnp.int32),
        ),
        scratch_types=[
            pltpu.VMEM((_EPP + 16,), jnp.int32),
            pltpu.VMEM((_EPP + 16,), jnp.int32),
            pltpu.VMEM((_LISTS,), jnp.int32),
            pltpu.VMEM((48,), jnp.int32),
            pltpu.VMEM((_OCAP,), jnp.int32),
        ],
    )
    def part_kernel(row_hbm, col_hbm, lists_hbm, counts_hbm, ovf_hbm,
                    rowv, colv, lists_v, cnt_v, ovf_v):
        c = lax.axis_index("c")
        s = lax.axis_index("s")
        p = c * 16 + s
        lane = lax.broadcasted_iota(jnp.int32, (16,), 0)
        neg1 = jnp.full((16,), -1, jnp.int32)
        zero16 = jnp.zeros((16,), jnp.int32)

        def _fill_lists(i, _):
            lists_v[pl.ds(i * 16, 16)] = neg1
            return 0

        lax.fori_loop(0, _LISTS // 16, _fill_lists, 0)

        def _fill_ovf(i, _):
            ovf_v[pl.ds(i * 16, 16)] = neg1
            return 0

        lax.fori_loop(0, _OCAP // 16, _fill_ovf, 0)
        for i in range(3):
            cnt_v[pl.ds(i * 16, 16)] = zero16

        base = p * _EPP
        pltpu.sync_copy(row_hbm.at[pl.ds(base, _EPP)], rowv.at[pl.ds(0, _EPP)])
        pltpu.sync_copy(col_hbm.at[pl.ds(base, _EPP)], colv.at[pl.ds(0, _EPP)])
        ngroups = (_EPP + 15) // 16
        tail = (ngroups - 1) * 16
        rt = rowv[pl.ds(tail, 16)]
        ct = colv[pl.ds(tail, 16)]
        nvalid = _EPP - tail
        rowv[pl.ds(tail, 16)] = jnp.where(lane < nvalid, rt, 1)
        colv[pl.ds(tail, 16)] = jnp.where(lane < nvalid, ct, 1)

        def _group(g, _):
            r16 = rowv[pl.ds(g * 16, 16)]
            c16 = colv[pl.ds(g * 16, 16)]
            b16 = (r16 * 13108) >> 22
            local16 = r16 - b16 * _RPT
            word16 = local16 * _PACK + c16
            gword16 = r16 * _PACK + c16
            bb16 = jnp.where(r16 != c16, b16, _NT)
            for l in range(16):
                bv = bb16[l]

                @pl.when(bv < _NT)
                def _append():
                    cntv = cnt_v[pl.ds(bv, 16)][0]
                    plsc.addupdate(cnt_v.at[pl.ds(bv, 16)],
                                   jnp.where(lane == 0, 1, 0))

                    @pl.when(cntv < _BCAP)
                    def _bucket():
                        wv = word16[l]
                        plsc.addupdate(
                            lists_v.at[pl.ds(bv * _BW + cntv, 16)],
                            jnp.where(lane == 0, wv + 1, 0))

                    @pl.when(cntv >= _BCAP)
                    def _overflow():
                        gv = gword16[l]
                        ov = cnt_v[pl.ds(_NT, 16)][0]
                        plsc.addupdate(cnt_v.at[pl.ds(_NT, 16)],
                                       jnp.where(lane == 0, 1, 0))
                        plsc.addupdate(ovf_v.at[pl.ds(ov, 16)],
                                       jnp.where(lane == 0, gv + 1, 0))
            return 0

        lax.fori_loop(0, ngroups, _group, 0)

        pltpu.sync_copy(lists_v, lists_hbm.at[pl.ds(p * _LISTS, _LISTS)])
        pltpu.sync_copy(cnt_v, counts_hbm.at[pl.ds(p * 48, 48)])
        pltpu.sync_copy(ovf_v, ovf_hbm.at[pl.ds(p * _OCAP, _OCAP)])

    return part_kernel(row, col)


def _sc_aggregate(x, lists, counts, ovf):
    mesh = plsc.VectorSubcoreMesh(core_axis_name="c", subcore_axis_name="s")

    @functools.partial(
        pl.kernel,
        mesh=mesh,
        out_type=jax.ShapeDtypeStruct((_NPAD, _D), jnp.float32),
        scratch_types=[
            pltpu.VMEM((_RPT + 16, _D), jnp.float32),
            pltpu.VMEM((2, _CH, _D), jnp.float32),
            pltpu.VMEM((2, _CH), jnp.int32),
            pltpu.VMEM((2, _CH + 16), jnp.int32),
            pltpu.VMEM((_NT * _BW,), jnp.int32),
            pltpu.VMEM((_NT * 48,), jnp.int32),
            pltpu.VMEM((_NT * _NCHUNK + 16,), jnp.int32),
            pltpu.SemaphoreType.DMA,
            pltpu.SemaphoreType.DMA,
            pltpu.SemaphoreType.DMA,
        ],
    )
    def agg_kernel(x_hbm, lists_hbm, counts_hbm, ovf_hbm, out_hbm,
                   acc, rows_v, cidx_v, dloc_v, lall_v, cnts_v, chk_v,
                   sem0, sem1, semL):
        c = lax.axis_index("c")
        s = lax.axis_index("s")
        w = c * 16 + s
        lane = lax.broadcasted_iota(jnp.int32, (16,), 0)
        zf16 = jnp.zeros((16,), jnp.float32)

        copies = []
        for p in range(_NT):
            copies.append(pltpu.async_copy(
                lists_hbm.at[pl.ds(p * _LISTS + w * _BW, _BW)],
                lall_v.at[pl.ds(p * _BW, _BW)], semL))
        pltpu.sync_copy(counts_hbm, cnts_v)

        def _zacc(r, _):
            for j in range(_D // 16):
                acc[r, pl.ds(j * 16, 16)] = zf16
            return 0

        lax.fori_loop(0, _RPT + 16, _zacc, 0)
        for i in range((_NT * _NCHUNK + 16) // 16):
            chk_v[pl.ds(i * 16, 16)] = jnp.zeros((16,), jnp.int32)
        for cp in copies:
            cp.wait()

        def _scan_p(p, m):
            cnt = cnts_v[pl.ds(p * 48 + w, 16)][0]

            def _scan_j(j, m2):
                @pl.when(j * _CH < cnt)
                def _app():
                    plsc.addupdate(chk_v.at[pl.ds(m2, 16)],
                                   jnp.where(lane == 0, p * _BW + j * _CH + 1,
                                             0))
                return m2 + jnp.where(j * _CH < cnt, 1, 0)

            return lax.fori_loop(0, _NCHUNK, _scan_j, m)

        m_total = lax.fori_loop(0, _NT, _scan_p, 0)

        def _fire(k, slot, cidx_s, dloc_s, rows_s, sem_s):
            off = chk_v[pl.ds(k, 16)][0] - 1
            for g in range(_CH // 16):
                w16 = lall_v[pl.ds(off + g * 16, 16)]
                sent = w16 < 0
                cidx_s[pl.ds(g * 16, 16)] = jnp.where(sent, 0,
                                                      w16 & (_PACK - 1))
                dloc_s[pl.ds(g * 16, 16)] = jnp.where(sent, _DUMMY, w16 >> 14)
            return pltpu.async_copy(x_hbm.at[cidx_s], rows_s, sem_s)

        def _adds(dloc_s, rows_s):
            for e0 in range(0, _CH, 2):
                dva = dloc_s[pl.ds(e0, 16)][0]
                dvb = dloc_s[pl.ds(e0 + 1, 16)][0]
                for jj in range(_D // 16):
                    plsc.addupdate(acc.at[dva, pl.ds(jj * 16, 16)],
                                   rows_s[e0, pl.ds(jj * 16, 16)])
                    plsc.addupdate(acc.at[dvb, pl.ds(jj * 16, 16)],
                                   rows_s[e0 + 1, pl.ds(jj * 16, 16)])

        cidx0, cidx1 = cidx_v.at[0], cidx_v.at[1]
        dloc0, dloc1 = dloc_v.at[0], dloc_v.at[1]
        rows0, rows1 = rows_v.at[0], rows_v.at[1]

        @pl.when(m_total > 0)
        def _run():
            _fire(0, 0, cidx0, dloc0, rows0, sem0)

            def _pair(t, _):
                @pl.when(2 * t + 1 < m_total)
                def _f1():
                    _fire(2 * t + 1, 1, cidx1, dloc1, rows1, sem1)

                @pl.when(2 * t < m_total)
                def _a0():
                    pltpu.make_async_copy(
                        x_hbm.at[cidx0], rows0, sem0).wait()
                    _adds(dloc0, rows0)

                @pl.when(2 * t + 2 < m_total)
                def _f0():
                    _fire(2 * t + 2, 0, cidx0, dloc0, rows0, sem0)

                @pl.when(2 * t + 1 < m_total)
                def _a1():
                    pltpu.make_async_copy(
                        x_hbm.at[cidx1], rows1, sem1).wait()
                    _adds(dloc1, rows1)
                return 0

            lax.fori_loop(0, (m_total + 1) >> 1, _pair, 0)

        def _ovf_producer(p, _):
            ocnt = cnts_v[pl.ds(p * 48 + _NT, 16)][0]

            @pl.when(ocnt > 0)
            def _scan():
                def _ogroup(g, _):
                    @pl.when(g * 16 < ocnt)
                    def _do():
                        pltpu.sync_copy(
                            ovf_hbm.at[pl.ds(p * _OCAP + g * 16, 16)],
                            dloc_v.at[1, pl.ds(0, 16)])
                        w16 = dloc_v[1, pl.ds(0, 16)]
                        r16 = w16 >> 14
                        c16 = w16 & (_PACK - 1)
                        mine = jnp.where(
                            (w16 >= 0) & (r16 >= w * _RPT)
                            & (r16 < w * _RPT + _RPT), 1, 0)
                        for l in range(16):
                            @pl.when(mine[l] == 1)
                            def _one():
                                cv = c16[l]
                                dv = r16[l] - w * _RPT
                                cidx_v[0, pl.ds(0, 16)] = jnp.where(
                                    lane == 0, cv, 0)
                                pltpu.async_copy(
                                    x_hbm.at[cidx_v.at[0, pl.ds(0, 16)]],
                                    rows_v.at[0, pl.ds(0, 16)], sem0).wait()
                                for jj in range(_D // 16):
                                    plsc.addupdate(
                                        acc.at[dv, pl.ds(jj * 16, 16)],
                                        rows_v[0, 0, pl.ds(jj * 16, 16)])
                    return 0

                lax.fori_loop(0, (_OCAP + 15) // 16, _ogroup, 0)
            return 0

        lax.fori_loop(0, _NT, _ovf_producer, 0)

        pltpu.sync_copy(acc.at[pl.ds(0, _RPT)],
                        out_hbm.at[pl.ds(w * _RPT, _RPT)])

    return agg_kernel(x, lists, counts, ovf)


def _tc_matmul_kernel(a_ref, x_ref, w_ref, r_ref, b_ref, o_ref):
    o_ref[...] = (
        jnp.dot(a_ref[...], w_ref[...], preferred_element_type=jnp.float32)
        + jnp.dot(x_ref[...], r_ref[...], preferred_element_type=jnp.float32)
        + b_ref[...]
    )


_BLK = 400


def _tc_matmul(agg, x, weight, root, bias):
    return pl.pallas_call(
        _tc_matmul_kernel,
        grid=(_N // _BLK,),
        in_specs=[
            pl.BlockSpec((_BLK, _D), lambda i: (i, 0)),
            pl.BlockSpec((_BLK, _D), lambda i: (i, 0)),
            pl.BlockSpec((_D, _D), lambda i: (0, 0)),
            pl.BlockSpec((_D, _D), lambda i: (0, 0)),
            pl.BlockSpec((1, _D), lambda i: (0, 0)),
        ],
        out_specs=pl.BlockSpec((_BLK, _D), lambda i: (i, 0)),
        out_shape=jax.ShapeDtypeStruct((_N, _D), jnp.float32),
    )(agg, x, weight, root, bias)


@jax.jit
def kernel(x, edge_index, weight, root, bias):
    row = edge_index[0]
    col = edge_index[1]
    lists, counts, ovf = _sc_partition(row, col)
    agg = _sc_aggregate(x, lists, counts, ovf)
    return _tc_matmul(agg[:_N], x, weight, root, bias.reshape(1, _D))


# speedup vs baseline: 1.0278x; 1.0130x over previous
import functools

import jax
import jax.numpy as jnp
from jax import lax
from jax.experimental import pallas as pl
from jax.experimental.pallas import tpu as pltpu
from jax.experimental.pallas import tpu_sc as plsc

_N = 10000
_E = 160000
_D = 256

_NT = 32
_RPT = 320
_NPAD = _NT * _RPT
_EPP = _E // _NT
_BCAP = 320
_BW = 336
_LISTS = _NT * _BW
_OCAP = _EPP + 24
_CH = 48
_NCHUNK = _BW // _CH
_DUMMY = _RPT
_PACK = 16384


def _sc_partition(row, col):
    mesh = plsc.VectorSubcoreMesh(core_axis_name="c", subcore_axis_name="s")

    @functools.partial(
        pl.kernel,
        mesh=mesh,
        out_type=(
            jax.ShapeDtypeStruct((_NT * _LISTS,), jnp.int32),
            jax.ShapeDtypeStruct((_NT * 48,), jnp.int32),
            jax.ShapeDtypeStruct((_NT * _OCAP,), jnp.int32),
        ),
        scratch_types=[
            pltpu.VMEM((_EPP + 16,), jnp.int32),
            pltpu.VMEM((_EPP + 16,), jnp.int32),
            pltpu.VMEM((_LISTS,), jnp.int32),
            pltpu.VMEM((48,), jnp.int32),
            pltpu.VMEM((_OCAP,), jnp.int32),
        ],
    )
    def part_kernel(row_hbm, col_hbm, lists_hbm, counts_hbm, ovf_hbm,
                    rowv, colv, lists_v, cnt_v, ovf_v):
        c = lax.axis_index("c")
        s = lax.axis_index("s")
        p = c * 16 + s
        lane = lax.broadcasted_iota(jnp.int32, (16,), 0)
        neg1 = jnp.full((16,), -1, jnp.int32)
        zero16 = jnp.zeros((16,), jnp.int32)

        def _fill_lists(i, _):
            lists_v[pl.ds(i * 16, 16)] = neg1
            return 0

        lax.fori_loop(0, _LISTS // 16, _fill_lists, 0)

        def _fill_ovf(i, _):
            ovf_v[pl.ds(i * 16, 16)] = neg1
            return 0

        lax.fori_loop(0, _OCAP // 16, _fill_ovf, 0)
        for i in range(3):
            cnt_v[pl.ds(i * 16, 16)] = zero16

        base = p * _EPP
        pltpu.sync_copy(row_hbm.at[pl.ds(base, _EPP)], rowv.at[pl.ds(0, _EPP)])
        pltpu.sync_copy(col_hbm.at[pl.ds(base, _EPP)], colv.at[pl.ds(0, _EPP)])
        ngroups = (_EPP + 15) // 16
        tail = (ngroups - 1) * 16
        rt = rowv[pl.ds(tail, 16)]
        ct = colv[pl.ds(tail, 16)]
        nvalid = _EPP - tail
        rowv[pl.ds(tail, 16)] = jnp.where(lane < nvalid, rt, 1)
        colv[pl.ds(tail, 16)] = jnp.where(lane < nvalid, ct, 1)

        def _group(g, _):
            r16 = rowv[pl.ds(g * 16, 16)]
            c16 = colv[pl.ds(g * 16, 16)]
            b16 = (r16 * 13108) >> 22
            local16 = r16 - b16 * _RPT
            word16 = local16 * _PACK + c16
            gword16 = r16 * _PACK + c16
            bb16 = jnp.where(r16 != c16, b16, _NT)
            for l in range(16):
                bv = bb16[l]

                @pl.when(bv < _NT)
                def _append():
                    cntv = cnt_v[pl.ds(bv, 16)][0]
                    plsc.addupdate(cnt_v.at[pl.ds(bv, 16)],
                                   jnp.where(lane == 0, 1, 0))

                    @pl.when(cntv < _BCAP)
                    def _bucket():
                        wv = word16[l]
                        plsc.addupdate(
                            lists_v.at[pl.ds(bv * _BW + cntv, 16)],
                            jnp.where(lane == 0, wv + 1, 0))

                    @pl.when(cntv >= _BCAP)
                    def _overflow():
                        gv = gword16[l]
                        ov = cnt_v[pl.ds(_NT, 16)][0]
                        plsc.addupdate(cnt_v.at[pl.ds(_NT, 16)],
                                       jnp.where(lane == 0, 1, 0))
                        plsc.addupdate(ovf_v.at[pl.ds(ov, 16)],
                                       jnp.where(lane == 0, gv + 1, 0))
            return 0

        lax.fori_loop(0, ngroups, _group, 0)

        pltpu.sync_copy(lists_v, lists_hbm.at[pl.ds(p * _LISTS, _LISTS)])
        pltpu.sync_copy(cnt_v, counts_hbm.at[pl.ds(p * 48, 48)])
        pltpu.sync_copy(ovf_v, ovf_hbm.at[pl.ds(p * _OCAP, _OCAP)])

    return part_kernel(row, col)


def _sc_aggregate(x, lists, counts, ovf):
    mesh = plsc.VectorSubcoreMesh(core_axis_name="c", subcore_axis_name="s")

    @functools.partial(
        pl.kernel,
        mesh=mesh,
        out_type=jax.ShapeDtypeStruct((_NPAD, _D), jnp.float32),
        scratch_types=[
            pltpu.VMEM((_RPT + 16, _D), jnp.float32),
            pltpu.VMEM((2, _CH, _D), jnp.float32),
            pltpu.VMEM((2, _CH), jnp.int32),
            pltpu.VMEM((2, _CH + 16), jnp.int32),
            pltpu.VMEM((_NT * _BW,), jnp.int32),
            pltpu.VMEM((_NT * 48,), jnp.int32),
            pltpu.VMEM((_NT * _NCHUNK + 16,), jnp.int32),
            pltpu.SemaphoreType.DMA,
            pltpu.SemaphoreType.DMA,
            pltpu.SemaphoreType.DMA,
        ],
    )
    def agg_kernel(x_hbm, lists_hbm, counts_hbm, ovf_hbm, out_hbm,
                   acc, rows_v, cidx_v, dloc_v, lall_v, cnts_v, chk_v,
                   sem0, sem1, semL):
        c = lax.axis_index("c")
        s = lax.axis_index("s")
        w = c * 16 + s
        lane = lax.broadcasted_iota(jnp.int32, (16,), 0)
        zf16 = jnp.zeros((16,), jnp.float32)

        copies = []
        for p in range(_NT):
            copies.append(pltpu.async_copy(
                lists_hbm.at[pl.ds(p * _LISTS + w * _BW, _BW)],
                lall_v.at[pl.ds(p * _BW, _BW)], semL))
        pltpu.sync_copy(counts_hbm, cnts_v)

        def _zacc(r, _):
            for j in range(_D // 16):
                acc[r, pl.ds(j * 16, 16)] = zf16
            return 0

        lax.fori_loop(0, _RPT + 16, _zacc, 0)
        for i in range((_NT * _NCHUNK + 16) // 16):
            chk_v[pl.ds(i * 16, 16)] = jnp.zeros((16,), jnp.int32)
        for cp in copies:
            cp.wait()

        def _scan_p(p, m):
            cnt = cnts_v[pl.ds(p * 48 + w, 16)][0]

            def _scan_j(j, m2):
                @pl.when(j * _CH < cnt)
                def _app():
                    plsc.addupdate(chk_v.at[pl.ds(m2, 16)],
                                   jnp.where(lane == 0, p * _BW + j * _CH + 1,
                                             0))
                return m2 + jnp.where(j * _CH < cnt, 1, 0)

            return lax.fori_loop(0, _NCHUNK, _scan_j, m)

        m_total = lax.fori_loop(0, _NT, _scan_p, 0)

        def _fire(k, slot, cidx_s, dloc_s, rows_s, sem_s):
            off = chk_v[pl.ds(k, 16)][0] - 1
            for g in range(_CH // 16):
                w16 = lall_v[pl.ds(off + g * 16, 16)]
                sent = w16 < 0
                cidx_s[pl.ds(g * 16, 16)] = jnp.where(sent, 0,
                                                      w16 & (_PACK - 1))
                dloc_s[pl.ds(g * 16, 16)] = jnp.where(sent, _DUMMY, w16 >> 14)
            return pltpu.async_copy(x_hbm.at[cidx_s], rows_s, sem_s)

        def _adds(dloc_s, rows_s):
            nb = _D // 16
            for e0 in range(0, _CH, 2):
                dva = dloc_s[pl.ds(e0, 16)][0]
                dvb = dloc_s[pl.ds(e0 + 1, 16)][0]
                va = [rows_s[e0, pl.ds(jj * 16, 16)] for jj in range(nb)]
                vb = [rows_s[e0 + 1, pl.ds(jj * 16, 16)] for jj in range(nb)]
                for jj in range(nb):
                    plsc.addupdate(acc.at[dva, pl.ds(jj * 16, 16)], va[jj])
                for jj in range(nb):
                    plsc.addupdate(acc.at[dvb, pl.ds(jj * 16, 16)], vb[jj])

        cidx0, cidx1 = cidx_v.at[0], cidx_v.at[1]
        dloc0, dloc1 = dloc_v.at[0], dloc_v.at[1]
        rows0, rows1 = rows_v.at[0], rows_v.at[1]

        @pl.when(m_total > 0)
        def _run():
            _fire(0, 0, cidx0, dloc0, rows0, sem0)

            def _pair(t, _):
                @pl.when(2 * t + 1 < m_total)
                def _f1():
                    _fire(2 * t + 1, 1, cidx1, dloc1, rows1, sem1)

                @pl.when(2 * t < m_total)
                def _a0():
                    pltpu.make_async_copy(
                        x_hbm.at[cidx0], rows0, sem0).wait()
                    _adds(dloc0, rows0)

                @pl.when(2 * t + 2 < m_total)
                def _f0():
                    _fire(2 * t + 2, 0, cidx0, dloc0, rows0, sem0)

                @pl.when(2 * t + 1 < m_total)
                def _a1():
                    pltpu.make_async_copy(
                        x_hbm.at[cidx1], rows1, sem1).wait()
                    _adds(dloc1, rows1)
                return 0

            lax.fori_loop(0, (m_total + 1) >> 1, _pair, 0)

        def _ovf_producer(p, _):
            ocnt = cnts_v[pl.ds(p * 48 + _NT, 16)][0]

            @pl.when(ocnt > 0)
            def _scan():
                def _ogroup(g, _):
                    @pl.when(g * 16 < ocnt)
                    def _do():
                        pltpu.sync_copy(
                            ovf_hbm.at[pl.ds(p * _OCAP + g * 16, 16)],
                            dloc_v.at[1, pl.ds(0, 16)])
                        w16 = dloc_v[1, pl.ds(0, 16)]
                        r16 = w16 >> 14
                        c16 = w16 & (_PACK - 1)
                        mine = jnp.where(
                            (w16 >= 0) & (r16 >= w * _RPT)
                            & (r16 < w * _RPT + _RPT), 1, 0)
                        for l in range(16):
                            @pl.when(mine[l] == 1)
                            def _one():
                                cv = c16[l]
                                dv = r16[l] - w * _RPT
                                cidx_v[0, pl.ds(0, 16)] = jnp.where(
                                    lane == 0, cv, 0)
                                pltpu.async_copy(
                                    x_hbm.at[cidx_v.at[0, pl.ds(0, 16)]],
                                    rows_v.at[0, pl.ds(0, 16)], sem0).wait()
                                for jj in range(_D // 16):
                                    plsc.addupdate(
                                        acc.at[dv, pl.ds(jj * 16, 16)],
                                        rows_v[0, 0, pl.ds(jj * 16, 16)])
                    return 0

                lax.fori_loop(0, (_OCAP + 15) // 16, _ogroup, 0)
            return 0

        lax.fori_loop(0, _NT, _ovf_producer, 0)

        pltpu.sync_copy(acc.at[pl.ds(0, _RPT)],
                        out_hbm.at[pl.ds(w * _RPT, _RPT)])

    return agg_kernel(x, lists, counts, ovf)


def _tc_matmul_kernel(a_ref, x_ref, w_ref, r_ref, b_ref, o_ref):
    o_ref[...] = (
        jnp.dot(a_ref[...], w_ref[...], preferred_element_type=jnp.float32)
        + jnp.dot(x_ref[...], r_ref[...], preferred_element_type=jnp.float32)
        + b_ref[...]
    )


_BLK = 400


def _tc_matmul(agg, x, weight, root, bias):
    return pl.pallas_call(
        _tc_matmul_kernel,
        grid=(_N // _BLK,),
        in_specs=[
            pl.BlockSpec((_BLK, _D), lambda i: (i, 0)),
            pl.BlockSpec((_BLK, _D), lambda i: (i, 0)),
            pl.BlockSpec((_D, _D), lambda i: (0, 0)),
            pl.BlockSpec((_D, _D), lambda i: (0, 0)),
            pl.BlockSpec((1, _D), lambda i: (0, 0)),
        ],
        out_specs=pl.BlockSpec((_BLK, _D), lambda i: (i, 0)),
        out_shape=jax.ShapeDtypeStruct((_N, _D), jnp.float32),
    )(agg, x, weight, root, bias)


@jax.jit
def kernel(x, edge_index, weight, root, bias):
    row = edge_index[0]
    col = edge_index[1]
    lists, counts, ovf = _sc_partition(row, col)
    agg = _sc_aggregate(x, lists, counts, ovf)
    return _tc_matmul(agg[:_N], x, weight, root, bias.reshape(1, _D))
